# baseline (device time: 186031 ns/iter reference)
import jax
import jax.numpy as jnp
from jax import lax
from jax.experimental import pallas as pl
from jax.experimental.pallas import tpu as pltpu

N_DEV = 16
SUBK = 2

RING = [0, 3, 2, 1, 5, 6, 7, 11, 10, 9, 13, 14, 15, 12, 8, 4]
RING_INV = [0, 3, 2, 1, 15, 4, 5, 6, 14, 9, 8, 7, 13, 10, 11, 12]


def kernel(x, W1, W2):
    m, d = x.shape
    n_pipes = 2 * SUBK
    sm = m // n_pipes

    def body(nbr_ref, x_ref, W1_ref, W2_ref, out_ref,
             xga, p, xs, xr, acs, acr):
        right = nbr_ref[0]
        left = nbr_ref[1]

        barrier_sem = pltpu.get_barrier_semaphore()
        for nbr in (left, right):
            pl.semaphore_signal(
                barrier_sem, inc=1,
                device_id=(nbr,), device_id_type=pl.DeviceIdType.MESH,
            )
        pl.semaphore_wait(barrier_sem, 2)

        tgts = [right, left] * SUBK

        def rowoff(i):
            return (i % 2) * (m // 2) + (i // 2) * sm

        def f(xc):
            h = jnp.dot(xc, W1_ref[...], preferred_element_type=jnp.float32)
            h = h * jax.nn.sigmoid(h)
            return jnp.dot(h, W2_ref[...], preferred_element_type=jnp.float32)

        def mk_x(i, s):
            return pltpu.make_async_remote_copy(
                src_ref=xga.at[i, s], dst_ref=xga.at[i, s + 1],
                send_sem=xs.at[i, s], recv_sem=xr.at[i, s],
                device_id=(tgts[i],), device_id_type=pl.DeviceIdType.MESH,
            )

        def mk_acc(i, s):
            return pltpu.make_async_remote_copy(
                src_ref=p.at[i, s], dst_ref=p.at[i, s + 1],
                send_sem=acs.at[i, s], recv_sem=acr.at[i, s],
                device_id=(tgts[i],), device_id_type=pl.DeviceIdType.MESH,
            )

        xd = [[None] * (N_DEV - 1) for _ in range(n_pipes)]
        ad = [[None] * N_DEV for _ in range(n_pipes)]

        for i in range(n_pipes):
            off = rowoff(i)
            xga[i, 0] = x_ref[off:off + sm, :]
            xd[i][0] = mk_x(i, 0)
            xd[i][0].start()
        for i in range(n_pipes):
            p[i, 0] = f(xga[i, 0])

        for s in range(1, N_DEV):
            for i in range(n_pipes):
                xd[i][s - 1].wait_recv()
                if s < N_DEV - 1:
                    xd[i][s] = mk_x(i, s)
                    xd[i][s].start()
                fi = f(xga[i, s])
                if s >= 2:
                    ad[i][s - 1].wait_recv()
                    p[i, s] = p[i, s] + fi
                else:
                    p[i, s] = fi
                ad[i][s] = mk_acc(i, s)
                ad[i][s].start()

        for i in range(n_pipes):
            ad[i][N_DEV - 1].wait_recv()
            off = rowoff(i)
            out_ref[off:off + sm, :] = p[i, N_DEV] + p[i, 0]

        for descs in xd + ad:
            for d_ in descs:
                if d_ is not None:
                    d_.wait_send()

    my = lax.axis_index("i")
    ring = jnp.asarray(RING, dtype=jnp.int32)
    ring_inv = jnp.asarray(RING_INV, dtype=jnp.int32)
    q = ring_inv[my]
    nbrs = jnp.stack([ring[(q + 1) % N_DEV], ring[(q + N_DEV - 1) % N_DEV]])

    return pl.pallas_call(
        body,
        out_shape=jax.ShapeDtypeStruct((m, d), jnp.float32),
        in_specs=[
            pl.BlockSpec(memory_space=pltpu.SMEM),
            pl.BlockSpec(memory_space=pltpu.VMEM),
            pl.BlockSpec(memory_space=pltpu.VMEM),
            pl.BlockSpec(memory_space=pltpu.VMEM),
        ],
        out_specs=pl.BlockSpec(memory_space=pltpu.VMEM),
        scratch_shapes=[
            pltpu.VMEM((n_pipes, N_DEV, sm, d), jnp.float32),
            pltpu.VMEM((n_pipes, N_DEV + 1, sm, d), jnp.float32),
            pltpu.SemaphoreType.DMA((n_pipes, N_DEV - 1)),
            pltpu.SemaphoreType.DMA((n_pipes, N_DEV - 1)),
            pltpu.SemaphoreType.DMA((n_pipes, N_DEV)),
            pltpu.SemaphoreType.DMA((n_pipes, N_DEV)),
        ],
        compiler_params=pltpu.CompilerParams(
            collective_id=0,
            vmem_limit_bytes=110 * 1024 * 1024,
        ),
    )(nbrs, x, W1, W2)


# device time: 184529 ns/iter; 1.0081x vs baseline; 1.0081x over previous
import jax
import jax.numpy as jnp
from jax import lax
from jax.experimental import pallas as pl
from jax.experimental.pallas import tpu as pltpu

N_DEV = 16
SUBK = 2


def kernel(x, W1, W2):
    m, d = x.shape
    n_pipes = 2 * SUBK
    sm = m // n_pipes

    def body(x_ref, W1_ref, W2_ref, out_ref,
             xga, p, xs, xr, acs, acr):
        my = lax.axis_index("i")
        right = (my + 1) % N_DEV
        left = (my + N_DEV - 1) % N_DEV

        barrier_sem = pltpu.get_barrier_semaphore()
        for nbr in (left, right):
            pl.semaphore_signal(
                barrier_sem, inc=1,
                device_id=(nbr,), device_id_type=pl.DeviceIdType.MESH,
            )
        pl.semaphore_wait(barrier_sem, 2)

        tgts = [right, left] * SUBK

        def rowoff(i):
            return (i % 2) * (m // 2) + (i // 2) * sm

        def f(xc):
            h = jnp.dot(xc, W1_ref[...], preferred_element_type=jnp.float32)
            h = h * jax.nn.sigmoid(h)
            return jnp.dot(h, W2_ref[...], preferred_element_type=jnp.float32)

        def mk_x(i, s):
            return pltpu.make_async_remote_copy(
                src_ref=xga.at[i, s], dst_ref=xga.at[i, s + 1],
                send_sem=xs.at[i, s], recv_sem=xr.at[i, s],
                device_id=(tgts[i],), device_id_type=pl.DeviceIdType.MESH,
            )

        def mk_acc(i, s):
            return pltpu.make_async_remote_copy(
                src_ref=p.at[i, s], dst_ref=p.at[i, s + 1],
                send_sem=acs.at[i, s], recv_sem=acr.at[i, s],
                device_id=(tgts[i],), device_id_type=pl.DeviceIdType.MESH,
            )

        xd = [[None] * (N_DEV - 1) for _ in range(n_pipes)]
        ad = [[None] * N_DEV for _ in range(n_pipes)]

        for i in range(n_pipes):
            off = rowoff(i)
            xga[i, 0] = x_ref[off:off + sm, :]
            xd[i][0] = mk_x(i, 0)
            xd[i][0].start()
        for i in range(n_pipes):
            p[i, 0] = f(xga[i, 0])

        for s in range(1, N_DEV):
            for i in range(n_pipes):
                xd[i][s - 1].wait_recv()
                if s < N_DEV - 1:
                    xd[i][s] = mk_x(i, s)
                    xd[i][s].start()
                fi = f(xga[i, s])
                if s >= 2:
                    ad[i][s - 1].wait_recv()
                    p[i, s] = p[i, s] + fi
                else:
                    p[i, s] = fi
                ad[i][s] = mk_acc(i, s)
                ad[i][s].start()

        for i in range(n_pipes):
            ad[i][N_DEV - 1].wait_recv()
            off = rowoff(i)
            out_ref[off:off + sm, :] = p[i, N_DEV] + p[i, 0]

        for descs in xd + ad:
            for d_ in descs:
                if d_ is not None:
                    d_.wait_send()

    return pl.pallas_call(
        body,
        out_shape=jax.ShapeDtypeStruct((m, d), jnp.float32),
        in_specs=[
            pl.BlockSpec(memory_space=pltpu.VMEM),
            pl.BlockSpec(memory_space=pltpu.VMEM),
            pl.BlockSpec(memory_space=pltpu.VMEM),
        ],
        out_specs=pl.BlockSpec(memory_space=pltpu.VMEM),
        scratch_shapes=[
            pltpu.VMEM((n_pipes, N_DEV, sm, d), jnp.float32),
            pltpu.VMEM((n_pipes, N_DEV + 1, sm, d), jnp.float32),
            pltpu.SemaphoreType.DMA((n_pipes, N_DEV - 1)),
            pltpu.SemaphoreType.DMA((n_pipes, N_DEV - 1)),
            pltpu.SemaphoreType.DMA((n_pipes, N_DEV)),
            pltpu.SemaphoreType.DMA((n_pipes, N_DEV)),
        ],
        compiler_params=pltpu.CompilerParams(
            collective_id=0,
            vmem_limit_bytes=110 * 1024 * 1024,
        ),
    )(x, W1, W2)
